# baseline (speedup 1.0000x reference)
import functools

import jax
import jax.numpy as jnp
from jax import lax
from jax.experimental import pallas as pl
from jax.experimental.pallas import tpu as pltpu
from jax.experimental.pallas import tpu_sc as plsc

_EMB = 64
_LSP = 208
_LAP = 32
_TOT = _LSP + _LAP
_NCHUNK = 3
_CHUNK = _TOT // _NCHUNK
_NC, _NS = 2, 16
_NW = _NC * _NS


def _sc_embed_means(tok3, emb_table):
    B = tok3.shape[0]
    per_w = B // _NW
    n_state_vregs = _LSP // 16
    G = 4
    ngrp = per_w // G
    mesh = plsc.VectorSubcoreMesh(
        core_axis_name="c", subcore_axis_name="s",
        num_cores=_NC, num_subcores=_NS)

    @functools.partial(
        pl.kernel,
        out_type=(jax.ShapeDtypeStruct((B, _EMB), jnp.float32),
                  jax.ShapeDtypeStruct((B, _EMB), jnp.float32)),
        mesh=mesh,
        compiler_params=pltpu.CompilerParams(
            needs_layout_passes=False, use_tc_tiling_on_sc=False),
        scratch_types=[
            pltpu.VMEM((2, G, _NCHUNK, _CHUNK), jnp.int32),
            pltpu.VMEM((2, G * _TOT, _EMB), jnp.float32),
            pltpu.VMEM((2, G, _EMB), jnp.float32),
            pltpu.VMEM((2, G, _EMB), jnp.float32),
            pltpu.SemaphoreType.DMA,
            pltpu.SemaphoreType.DMA,
            pltpu.SemaphoreType.DMA,
            pltpu.SemaphoreType.DMA,
        ],
    )
    def k(tok_hbm, e_hbm, out_s_hbm, out_a_hbm,
          idx_v, rows_v, os_v, oa_v, gsem0, gsem1, osem0, osem1):
        wid = lax.axis_index("s") * _NC + lax.axis_index("c")
        base = wid * per_w
        gsems = (gsem0, gsem1)
        osems = (osem0, osem1)

        def issue(g, buf):
            idx4 = idx_v.at[buf]
            pltpu.sync_copy(tok_hbm.at[pl.ds(base + g * G, G)], idx4)
            for e in range(G):
                for c in range(_NCHUNK):
                    pltpu.async_copy(
                        e_hbm.at[idx4.at[e, c]],
                        rows_v.at[buf, pl.ds(e * _TOT + c * _CHUNK, _CHUNK)],
                        gsems[buf])

        def out_descs(g, buf):
            dst = pl.ds(base + g * G, G)
            return (pltpu.make_async_copy(os_v.at[buf], out_s_hbm.at[dst],
                                          osems[buf]),
                    pltpu.make_async_copy(oa_v.at[buf], out_a_hbm.at[dst],
                                          osems[buf]))

        def consume(g, buf):
            idx4 = idx_v.at[buf]
            for e in range(G):
                for c in range(_NCHUNK):
                    pltpu.make_async_copy(
                        e_hbm.at[idx4.at[e, c]],
                        rows_v.at[buf, pl.ds(e * _TOT + c * _CHUNK, _CHUNK)],
                        gsems[buf]).wait()

            @pl.when(g >= 2)
            def _():
                for d in out_descs(g - 2, buf):
                    d.wait()

            for e in range(G):
                cs = jnp.zeros((16,), jnp.int32)
                ca = jnp.zeros((16,), jnp.int32)
                for kc in range(_TOT // 16):
                    t = idx4[e, kc // 5, pl.ds((kc % 5) * 16, 16)]
                    pc = plsc.all_reduce_population_count(t != 0)
                    if kc < n_state_vregs:
                        cs = cs + pc
                    else:
                        ca = ca + pc
                inv_s = 1.0 / jnp.maximum(cs.astype(jnp.float32), 1.0)
                inv_a = 1.0 / jnp.maximum(ca.astype(jnp.float32), 1.0)

                def rbody(r, acc):
                    return tuple(
                        acc[j] + rows_v[buf, r, pl.ds(16 * j, 16)]
                        for j in range(_EMB // 16))

                z4 = tuple(jnp.zeros((16,), jnp.float32)
                           for _ in range(_EMB // 16))
                lo = e * _TOT
                acc_s = lax.fori_loop(lo, lo + _LSP, rbody, z4, unroll=4)
                acc_a = lax.fori_loop(lo + _LSP, lo + _TOT, rbody, z4,
                                      unroll=4)
                for j in range(_EMB // 16):
                    os_v[buf, e, pl.ds(16 * j, 16)] = acc_s[j] * inv_s
                    oa_v[buf, e, pl.ds(16 * j, 16)] = acc_a[j] * inv_a

            for d in out_descs(g, buf):
                d.start()

        issue(0, 0)
        nit = ngrp // 2

        def body(i, carry):
            g0 = 2 * i
            issue(g0 + 1, 1)
            consume(g0, 0)

            @pl.when(i < nit - 1)
            def _():
                issue(g0 + 2, 0)

            consume(g0 + 1, 1)
            return carry

        lax.fori_loop(0, nit, body, 0)
        for d in out_descs(ngrp - 2, 0):
            d.wait()
        for d in out_descs(ngrp - 1, 1):
            d.wait()

    return k(tok3, emb_table)


def _tc_mlp(ms, ma, w1t, b1, w2t, b2, w3a, w3b, b3, w4, b4):
    B, H = ms.shape[0], w1t.shape[1]
    BS = 2048

    def body(ms_ref, ma_ref, w1_ref, b1_ref, w2_ref, b2_ref,
             w3a_ref, w3b_ref, b3_ref, w4_ref, b4_ref, out_ref):
        hs = jnp.tanh(jnp.dot(ms_ref[...], w1_ref[...],
                              preferred_element_type=jnp.float32) + b1_ref[...])
        ha = jnp.tanh(jnp.dot(ma_ref[...], w2_ref[...],
                              preferred_element_type=jnp.float32) + b2_ref[...])
        hs = jnp.maximum(hs, 0.0)
        ha = jnp.maximum(ha, 0.0)
        h = (jnp.dot(hs, w3a_ref[...], preferred_element_type=jnp.float32)
             + jnp.dot(ha, w3b_ref[...], preferred_element_type=jnp.float32)
             + b3_ref[...])
        h = jnp.maximum(h, 0.0)
        out_ref[...] = (jnp.sum(h * w4_ref[...], axis=1, keepdims=True)
                        + b4_ref[...])

    return pl.pallas_call(
        body,
        grid=(B // BS,),
        in_specs=[
            pl.BlockSpec((BS, _EMB), lambda i: (i, 0)),
            pl.BlockSpec((BS, _EMB), lambda i: (i, 0)),
            pl.BlockSpec((_EMB, H), lambda i: (0, 0)),
            pl.BlockSpec((1, H), lambda i: (0, 0)),
            pl.BlockSpec((_EMB, H), lambda i: (0, 0)),
            pl.BlockSpec((1, H), lambda i: (0, 0)),
            pl.BlockSpec((H, H), lambda i: (0, 0)),
            pl.BlockSpec((H, H), lambda i: (0, 0)),
            pl.BlockSpec((1, H), lambda i: (0, 0)),
            pl.BlockSpec((1, H), lambda i: (0, 0)),
            pl.BlockSpec((1, 1), lambda i: (0, 0)),
        ],
        out_specs=pl.BlockSpec((BS, 1), lambda i: (i, 0)),
        out_shape=jax.ShapeDtypeStruct((B, 1), jnp.float32),
    )(ms, ma, w1t, b1, w2t, b2, w3a, w3b, b3, w4, b4)


def kernel(state_tokens, state_lengths, action_tokens, action_lengths,
           E, W1, b1, W2, b2, W3, b3, W4, b4):
    del state_lengths, action_lengths
    B = state_tokens.shape[0]
    H = W1.shape[0]
    st = state_tokens.astype(jnp.int32)
    at = action_tokens.astype(jnp.int32)
    tok = jnp.concatenate([
        jnp.pad(st, ((0, 0), (0, _LSP - st.shape[1]))),
        jnp.pad(at, ((0, 0), (0, _LAP - at.shape[1]))),
    ], axis=1).reshape(B, _NCHUNK, _CHUNK)
    ms, ma = _sc_embed_means(tok, E)
    w3t = W3.T
    return _tc_mlp(ms, ma,
                   W1.T, b1.reshape(1, H),
                   W2.T, b2.reshape(1, H),
                   w3t[:H], w3t[H:], b3.reshape(1, H),
                   W4, b4.reshape(1, 1))


# speedup vs baseline: 6.7760x; 6.7760x over previous
import functools

import jax
import jax.numpy as jnp
from jax import lax
from jax.experimental import pallas as pl
from jax.experimental.pallas import tpu as pltpu
from jax.experimental.pallas import tpu_sc as plsc

_EMB = 64
_EMBW = _EMB
_LS = 200
_LA = 20
_GCH = 80
_NC, _NS = 2, 16
_NW = _NC * _NS


def _sc_embed_sums(st2, at2, table):
    B = st2.shape[0] * _GCH // _LS
    per_w = B // _NW
    G = 4
    ngrp = per_w // G
    nsch = G * _LS // _GCH
    nach = G * _LA // _GCH
    srows = G * _LS
    mesh = plsc.VectorSubcoreMesh(
        core_axis_name="c", subcore_axis_name="s",
        num_cores=_NC, num_subcores=_NS)

    @functools.partial(
        pl.kernel,
        out_type=(jax.ShapeDtypeStruct((B, _EMB), jnp.float32),
                  jax.ShapeDtypeStruct((B, _EMB), jnp.float32)),
        mesh=mesh,
        compiler_params=pltpu.CompilerParams(
            needs_layout_passes=False, use_tc_tiling_on_sc=False),
        scratch_types=[
            pltpu.VMEM((2, nsch, _GCH), jnp.int32),
            pltpu.VMEM((2, nach, _GCH), jnp.int32),
            pltpu.VMEM((2, G * (_LS + _LA), _EMBW), jnp.float32),
            pltpu.VMEM((2, G, _EMB), jnp.float32),
            pltpu.VMEM((2, G, _EMB), jnp.float32),
            pltpu.SemaphoreType.DMA,
            pltpu.SemaphoreType.DMA,
            pltpu.SemaphoreType.DMA,
            pltpu.SemaphoreType.DMA,
        ],
    )
    def k(st_hbm, at_hbm, e_hbm, out_s_hbm, out_a_hbm,
          sidx_v, aidx_v, rows_v, os_v, oa_v, gsem0, gsem1, osem0, osem1):
        wid = lax.axis_index("s") * _NC + lax.axis_index("c")
        base = wid * per_w
        sbase = wid * (per_w * _LS // _GCH)
        abase = wid * (per_w * _LA // _GCH)
        gsems = (gsem0, gsem1)
        osems = (osem0, osem1)

        def gather_descs(buf):
            ds_ = []
            for c in range(nsch):
                ds_.append(pltpu.make_async_copy(
                    e_hbm.at[sidx_v.at[buf, c]],
                    rows_v.at[buf, pl.ds(c * _GCH, _GCH)],
                    gsems[buf]))
            for c in range(nach):
                ds_.append(pltpu.make_async_copy(
                    e_hbm.at[aidx_v.at[buf, c]],
                    rows_v.at[buf, pl.ds(srows + c * _GCH, _GCH)],
                    gsems[buf]))
            return ds_

        def issue(g, buf):
            pltpu.sync_copy(st_hbm.at[pl.ds(sbase + g * nsch, nsch)],
                            sidx_v.at[buf])
            pltpu.sync_copy(at_hbm.at[pl.ds(abase + g * nach, nach)],
                            aidx_v.at[buf])
            for d in gather_descs(buf):
                d.start()

        def out_descs(g, buf):
            dst = pl.ds(base + g * G, G)
            return (pltpu.make_async_copy(os_v.at[buf], out_s_hbm.at[dst],
                                          osems[buf]),
                    pltpu.make_async_copy(oa_v.at[buf], out_a_hbm.at[dst],
                                          osems[buf]))

        def consume(g, buf):
            for d in gather_descs(buf):
                d.wait()

            @pl.when(g >= 2)
            def _():
                for d in out_descs(g - 2, buf):
                    d.wait()

            def rbody(r, acc):
                return tuple(
                    acc[j] + rows_v[buf, r, pl.ds(16 * j, 16)]
                    for j in range(_EMBW // 16))

            for e in range(G):
                z4 = tuple(jnp.zeros((16,), jnp.float32) for _ in range(4))
                acc_s = lax.fori_loop(e * _LS, (e + 1) * _LS, rbody, z4,
                                      unroll=4)
                acc_a = lax.fori_loop(srows + e * _LA,
                                      srows + (e + 1) * _LA, rbody, z4,
                                      unroll=4)
                for j in range(4):
                    os_v[buf, e, pl.ds(16 * j, 16)] = acc_s[j]
                    oa_v[buf, e, pl.ds(16 * j, 16)] = acc_a[j]

            for d in out_descs(g, buf):
                d.start()

        issue(0, 0)
        nit = ngrp // 2

        def body(i, carry):
            g0 = 2 * i
            issue(g0 + 1, 1)
            consume(g0, 0)

            @pl.when(i < nit - 1)
            def _():
                issue(g0 + 2, 0)

            consume(g0 + 1, 1)
            return carry

        lax.fori_loop(0, nit, body, 0)
        for d in out_descs(ngrp - 2, 0):
            d.wait()
        for d in out_descs(ngrp - 1, 1):
            d.wait()

    return k(st2, at2, table)


def _tc_head(ss, sa, stok, atok, w1t, b1, w2t, b2, w3a, w3b, b3, w4, b4):
    B, H = ss.shape[0], w1t.shape[1]
    BS = 2048

    def body(ss_ref, sa_ref, st_ref, at_ref, w1_ref, b1_ref, w2_ref, b2_ref,
             w3a_ref, w3b_ref, b3_ref, w4_ref, b4_ref, out_ref):
        cs = jnp.sum((st_ref[...] != 0).astype(jnp.float32), axis=1,
                     keepdims=True)
        ca = jnp.sum((at_ref[...] != 0).astype(jnp.float32), axis=1,
                     keepdims=True)
        ms = ss_ref[...] / jnp.maximum(cs, 1.0)
        ma = sa_ref[...] / jnp.maximum(ca, 1.0)
        hs = jnp.tanh(jnp.dot(ms, w1_ref[...],
                              preferred_element_type=jnp.float32) + b1_ref[...])
        ha = jnp.tanh(jnp.dot(ma, w2_ref[...],
                              preferred_element_type=jnp.float32) + b2_ref[...])
        hs = jnp.maximum(hs, 0.0)
        ha = jnp.maximum(ha, 0.0)
        h = (jnp.dot(hs, w3a_ref[...], preferred_element_type=jnp.float32)
             + jnp.dot(ha, w3b_ref[...], preferred_element_type=jnp.float32)
             + b3_ref[...])
        h = jnp.maximum(h, 0.0)
        out_ref[...] = (jnp.sum(h * w4_ref[...], axis=1, keepdims=True)
                        + b4_ref[...])

    return pl.pallas_call(
        body,
        grid=(B // BS,),
        in_specs=[
            pl.BlockSpec((BS, _EMB), lambda i: (i, 0)),
            pl.BlockSpec((BS, _EMB), lambda i: (i, 0)),
            pl.BlockSpec((BS, _LS), lambda i: (i, 0)),
            pl.BlockSpec((BS, _LA), lambda i: (i, 0)),
            pl.BlockSpec((_EMB, H), lambda i: (0, 0)),
            pl.BlockSpec((1, H), lambda i: (0, 0)),
            pl.BlockSpec((_EMB, H), lambda i: (0, 0)),
            pl.BlockSpec((1, H), lambda i: (0, 0)),
            pl.BlockSpec((H, H), lambda i: (0, 0)),
            pl.BlockSpec((H, H), lambda i: (0, 0)),
            pl.BlockSpec((1, H), lambda i: (0, 0)),
            pl.BlockSpec((1, H), lambda i: (0, 0)),
            pl.BlockSpec((1, 1), lambda i: (0, 0)),
        ],
        out_specs=pl.BlockSpec((BS, 1), lambda i: (i, 0)),
        out_shape=jax.ShapeDtypeStruct((B, 1), jnp.float32),
    )(ss, sa, stok, atok, w1t, b1, w2t, b2, w3a, w3b, b3, w4, b4)


def kernel(state_tokens, state_lengths, action_tokens, action_lengths,
           E, W1, b1, W2, b2, W3, b3, W4, b4):
    del state_lengths, action_lengths
    B = state_tokens.shape[0]
    H = W1.shape[0]
    st = state_tokens.astype(jnp.int32)
    at = action_tokens.astype(jnp.int32)
    ss, sa = _sc_embed_sums(st.reshape(B * _LS // _GCH, _GCH),
                            at.reshape(B * _LA // _GCH, _GCH), E)
    w3t = W3.T
    return _tc_head(ss, sa, st, at,
                    W1.T, b1.reshape(1, H),
                    W2.T, b2.reshape(1, H),
                    w3t[:H], w3t[H:], b3.reshape(1, H),
                    W4, b4.reshape(1, 1))
